# Initial kernel scaffold; baseline (speedup 1.0000x reference)
import jax
import jax.numpy as jnp
from jax.experimental import pallas as pl


def kernel(x, conv1_w, conv1_b, conv2_w, conv2_b, conv3_w, conv3_b, w1t, b1, wat, ba, wit, bi, wft, bf):
    raise NotImplementedError("write your pallas kernel here")


# speedup vs baseline: 17.2706x; 17.2706x over previous
import jax
import jax.numpy as jnp
from jax.experimental import pallas as pl
from jax.experimental.pallas import tpu as pltpu

BB = 2


def _conv_relu_pool(xp, w_ref, b_ref, n, h, cin, cout):
    acc = jnp.zeros((n * h * h, cout), jnp.float32)
    for ky in range(3):
        for kx in range(3):
            patch = xp[:, ky:ky + h, kx:kx + h, :].reshape(n * h * h, cin)
            acc = acc + jnp.dot(patch, w_ref[ky, kx],
                                preferred_element_type=jnp.float32)
    y = jnp.maximum(acc + b_ref[...], 0.0)
    y = y.reshape(n, h // 2, 2, h, cout)
    y = jnp.max(y, axis=2)
    y = y.reshape(n, h // 2, h // 2, 2, cout)
    return jnp.max(y, axis=3)


def _fused_kernel(x_ref, w1_ref, b1_ref, w2_ref, b2_ref, w3_ref, b3_ref,
                  fw1_ref, fb1_ref, wa_ref, ba_ref, wi_ref, bi_ref,
                  wf_ref, bf_ref, logits_ref, concepts_ref):
    n = x_ref.shape[0]
    cin_pad = w1_ref.shape[2]
    x = jnp.transpose(x_ref[...], (0, 2, 3, 1))
    x = jnp.pad(x, ((0, 0), (1, 1), (1, 1), (0, cin_pad - x.shape[3])))

    y = _conv_relu_pool(x, w1_ref, b1_ref, n, 64, cin_pad, 8)
    y = jnp.pad(y, ((0, 0), (1, 1), (1, 1), (0, 0)))
    y = _conv_relu_pool(y, w2_ref, b2_ref, n, 32, 8, 16)
    y = jnp.pad(y, ((0, 0), (1, 1), (1, 1), (0, 0)))
    y = _conv_relu_pool(y, w3_ref, b3_ref, n, 16, 16, 32)

    feats = y.reshape(n, 2048)
    h = jnp.dot(feats, fw1_ref[...], preferred_element_type=jnp.float32)
    h = jnp.maximum(h + fb1_ref[...], 0.0)
    concepts = jnp.dot(h, wa_ref[...],
                       preferred_element_type=jnp.float32) + ba_ref[...]
    z = jnp.dot(concepts, wi_ref[...],
                preferred_element_type=jnp.float32) + bi_ref[...]
    z = jnp.maximum(z, 0.0) + h
    logits_ref[0] = jnp.dot(z, wf_ref[...],
                            preferred_element_type=jnp.float32) + bf_ref[...]
    concepts_ref[0] = concepts


def kernel(x, conv1_w, conv1_b, conv2_w, conv2_b, conv3_w, conv3_b,
           w1t, b1, wat, ba, wit, bi, wft, bf):
    B = x.shape[0]
    class_pad = wft.shape[1]
    attr_pad = wat.shape[1]
    cin_pad = conv1_w.shape[2]

    def _whole(a):
        return pl.BlockSpec(a.shape, lambda i: (0,) * a.ndim)

    G = B // BB
    logits_pad, concepts_pad = pl.pallas_call(
        _fused_kernel,
        out_shape=(
            jax.ShapeDtypeStruct((G, BB, class_pad), jnp.float32),
            jax.ShapeDtypeStruct((G, BB, attr_pad), jnp.float32),
        ),
        grid=(G,),
        in_specs=[
            pl.BlockSpec((BB, x.shape[1], 64, 64), lambda i: (i, 0, 0, 0)),
            _whole(conv1_w), _whole(conv1_b),
            _whole(conv2_w), _whole(conv2_b),
            _whole(conv3_w), _whole(conv3_b),
            _whole(w1t), _whole(b1),
            _whole(wat), _whole(ba),
            _whole(wit), _whole(bi),
            _whole(wft), _whole(bf),
        ],
        out_specs=(
            pl.BlockSpec((1, BB, class_pad), lambda i: (i, 0, 0)),
            pl.BlockSpec((1, BB, attr_pad), lambda i: (i, 0, 0)),
        ),
        compiler_params=pltpu.CompilerParams(
            dimension_semantics=("parallel",),
            vmem_limit_bytes=64 * 1024 * 1024,
        ),
    )(x, conv1_w, conv1_b, conv2_w, conv2_b, conv3_w, conv3_b,
      w1t, b1, wat, ba, wit, bi, wft, bf)

    n_classes = 200
    n_attr = 312
    logits_pad = logits_pad.reshape(B, class_pad)
    concepts_pad = concepts_pad.reshape(B, attr_pad)
    return logits_pad[:, :n_classes], concepts_pad[:, :n_attr]


# speedup vs baseline: 84.9278x; 4.9175x over previous
import jax
import jax.numpy as jnp
from jax.experimental import pallas as pl
from jax.experimental.pallas import tpu as pltpu

BB = 8


def _fused_kernel(x_ref, e_ref, t1_ref, b1_ref, t2_ref, b2_ref, t3_ref, b3_ref,
                  p1e_ref, p1o_ref, p2e_ref, p2o_ref, p3e_ref, p3o_ref,
                  fw1_ref, fb1_ref, wa_ref, ba_ref, wi_ref, bi_ref,
                  wf_ref, bf_ref, logits_ref, concepts_ref):
    n = x_ref.shape[0]
    f32 = jnp.float32

    acc = jnp.zeros((n * 64, 528), f32)
    for c in range(3):
        acc = acc + jnp.dot(x_ref[:, c].reshape(n * 64, 64), e_ref[c],
                            preferred_element_type=f32)
    xp = jnp.pad(acc.reshape(n, 64, 528), ((0, 0), (1, 1), (0, 0)))

    def conv(xp, t_ref, b_ref, h, wlanes):
        a = jnp.zeros((n * h, wlanes), f32)
        for ky in range(3):
            sl = xp[:, ky:ky + h, :].reshape(n * h, xp.shape[2])
            a = a + jnp.dot(sl, t_ref[ky], preferred_element_type=f32)
        return jnp.maximum(a + b_ref[...], 0.0)

    def pool(a, pe_ref, po_ref, n, h):
        m = jnp.maximum(jnp.dot(a, pe_ref[...], preferred_element_type=f32),
                        jnp.dot(a, po_ref[...], preferred_element_type=f32))
        m = m.reshape(n, h // 2, 2, m.shape[1])
        return jnp.max(m, axis=2)

    y = conv(xp, t1_ref, b1_ref, 64, 512)
    y = pool(y, p1e_ref, p1o_ref, n, 64)
    y = jnp.pad(y, ((0, 0), (1, 1), (8, 8)))

    y = conv(y, t2_ref, b2_ref, 32, 512)
    y = pool(y, p2e_ref, p2o_ref, n, 32)
    y = jnp.pad(y, ((0, 0), (1, 1), (16, 16)))

    y = conv(y, t3_ref, b3_ref, 16, 512)
    y = pool(y, p3e_ref, p3o_ref, n, 16)

    h = jnp.zeros((n, 128), f32)
    for yy in range(8):
        h = h + jnp.dot(y[:, yy, :], fw1_ref[yy], preferred_element_type=f32)
    h = jnp.maximum(h + fb1_ref[...], 0.0)
    concepts = jnp.dot(h, wa_ref[...],
                       preferred_element_type=f32) + ba_ref[...]
    z = jnp.dot(concepts, wi_ref[...],
                preferred_element_type=f32) + bi_ref[...]
    z = jnp.maximum(z, 0.0) + h
    logits_ref[...] = jnp.dot(z, wf_ref[...],
                              preferred_element_type=f32) + bf_ref[...]
    concepts_ref[...] = concepts


def _toeplitz(w_hwio, cin, cout, wout, wp):
    eye = jnp.eye(wout, dtype=jnp.float32)
    rows = []
    for ky in range(3):
        t = jnp.zeros((wp * cin, wout * cout), jnp.float32)
        for kx in range(3):
            blk = jnp.kron(eye, w_hwio[ky, kx, :cin])
            t = t.at[kx * cin:kx * cin + wout * cin].add(blk)
        rows.append(t)
    return jnp.stack(rows)


def _pool_perms(w, c):
    eye_w = jnp.eye(w, dtype=jnp.float32)
    eye_c = jnp.eye(c, dtype=jnp.float32)
    return (jnp.kron(eye_w[:, 0::2], eye_c), jnp.kron(eye_w[:, 1::2], eye_c))


def kernel(x, conv1_w, conv1_b, conv2_w, conv2_b, conv3_w, conv3_b,
           w1t, b1, wat, ba, wit, bi, wft, bf):
    B = x.shape[0]
    f32 = jnp.float32

    e = jnp.zeros((3, 64, 528), f32)
    xs = jnp.arange(64)
    for c in range(3):
        e = e.at[c, xs, (xs + 1) * 8 + c].set(1.0)

    t1 = _toeplitz(conv1_w, 8, 8, 64, 66)
    t2 = _toeplitz(conv2_w, 8, 16, 32, 34)
    t3 = _toeplitz(conv3_w, 16, 32, 16, 18)
    b1r = jnp.tile(conv1_b, (1, 64))
    b2r = jnp.tile(conv2_b, (1, 32))
    b3r = jnp.tile(conv3_b, (1, 16))
    p1e, p1o = _pool_perms(64, 8)
    p2e, p2o = _pool_perms(32, 16)
    p3e, p3o = _pool_perms(16, 32)
    fw1 = w1t.reshape(8, 256, 128)

    def _whole(a):
        return pl.BlockSpec(a.shape, lambda i: (0,) * a.ndim)

    args = (e, t1, b1r, t2, b2r, t3, b3r,
            p1e, p1o, p2e, p2o, p3e, p3o,
            fw1, b1, wat, ba, wit, bi, wft, bf)

    logits_pad, concepts_pad = pl.pallas_call(
        _fused_kernel,
        out_shape=(
            jax.ShapeDtypeStruct((B, wft.shape[1]), f32),
            jax.ShapeDtypeStruct((B, wat.shape[1]), f32),
        ),
        grid=(B // BB,),
        in_specs=[pl.BlockSpec((BB, x.shape[1], 64, 64), lambda i: (i, 0, 0, 0))]
                 + [_whole(a) for a in args],
        out_specs=(
            pl.BlockSpec((BB, wft.shape[1]), lambda i: (i, 0)),
            pl.BlockSpec((BB, wat.shape[1]), lambda i: (i, 0)),
        ),
        compiler_params=pltpu.CompilerParams(
            dimension_semantics=("parallel",),
            vmem_limit_bytes=64 * 1024 * 1024,
        ),
    )(x, *args)

    return logits_pad[:, :200], concepts_pad[:, :312]
